# baseline (speedup 1.0000x reference)
import functools

import jax
import jax.numpy as jnp
from jax import lax
from jax.experimental import pallas as pl
from jax.experimental.pallas import tpu as pltpu
from jax.experimental.pallas import tpu_sc as plsc

_I = 1000
_D = 26
_B = 16384
_PA = (0, 2, 4, 6, 8, 10, 12, 14, 16, 18, 20, 22, 24, 0, 1, 4)
_PB = (1, 3, 5, 7, 9, 11, 13, 15, 17, 19, 21, 23, 25, 2, 3, 6)
_P = 16
_C = 8
_PC = _P // _C

_NC = 2
_NS = 16
_NW = _NC * _NS
_BPW = _B // _NW
_G = _BPW // 16
_PB_W = _P * _BPW
_CPW = _PC * _BPW

_mesh = plsc.VectorSubcoreMesh(
    core_axis_name="c", subcore_axis_name="s", num_cores=_NC, num_subcores=_NS
)


@functools.partial(
    pl.kernel,
    mesh=_mesh,
    out_type=(
        jax.ShapeDtypeStruct((_B,), jnp.float32),
        jax.ShapeDtypeStruct((_B * _P,), jnp.int32),
    ),
    compiler_params=pltpu.CompilerParams(needs_layout_passes=False),
    scratch_types=[
        pltpu.VMEM((_D, _BPW), jnp.int32),
        pltpu.VMEM((_D * _I,), jnp.float32),
        pltpu.VMEM((_PB_W,), jnp.int32),
        pltpu.VMEM((_BPW,), jnp.float32),
        pltpu.VMEM((16,), jnp.float32),
    ],
)
def _gam_phase1(xT, t0, singles, acc_out, pidx_out, x_v, sing_v, pidx_v,
                acc_v, t0_v):
    wid = lax.axis_index("s") * _NC + lax.axis_index("c")
    base = wid * _BPW
    pltpu.sync_copy(xT.at[:, pl.ds(base, _BPW)], x_v)
    pltpu.sync_copy(singles, sing_v)
    pltpu.sync_copy(t0, t0_v)

    def idx_body(g, carry):
        s0 = pl.multiple_of(g * 16, 16)
        acc = t0_v[...]
        for d in range(_D):
            iv = x_v[d, pl.ds(s0, 16)]
            acc = acc + plsc.load_gather(sing_v, [iv + d * _I])
        acc_v[pl.ds(s0, 16)] = acc
        f0 = pl.multiple_of(g * (_PC * 16), 16)
        for p in range(_P):
            i = x_v[_PA[p], pl.ds(s0, 16)]
            j = x_v[_PB[p], pl.ds(s0, 16)]
            pidx_v[pl.ds((p // _PC) * _CPW + f0 + (p % _PC) * 16, 16)] = (
                i * _I + j + (p % _PC) * (_I * _I)
            )
        return carry

    lax.fori_loop(0, _G, idx_body, 0)

    pltpu.sync_copy(acc_v, acc_out.at[pl.ds(base, _BPW)])
    for c in range(_C):
        pltpu.sync_copy(
            pidx_v.at[pl.ds(c * _CPW, _CPW)],
            pidx_out.at[pl.ds((c * _NW + wid) * _CPW, _CPW)],
        )


@functools.partial(
    pl.kernel,
    mesh=_mesh,
    out_type=jax.ShapeDtypeStruct((_B,), jnp.float32),
    compiler_params=pltpu.CompilerParams(needs_layout_passes=False),
    scratch_types=[
        pltpu.VMEM((_CPW,), jnp.int32),
        pltpu.VMEM((_CPW,), jnp.float32),
        pltpu.VMEM((_BPW,), jnp.float32),
        pltpu.SemaphoreType.DMA,
    ],
)
def _gam_chunk(chunk_flat, pidx_c, acc_in, out, pidx_v, pval_v, acc_v, sem):
    wid = lax.axis_index("s") * _NC + lax.axis_index("c")
    base = wid * _BPW
    pltpu.sync_copy(pidx_c.at[pl.ds(wid * _CPW, _CPW)], pidx_v)
    gather = pltpu.async_copy(chunk_flat.at[pidx_v], pval_v, sem)
    pltpu.sync_copy(acc_in.at[pl.ds(base, _BPW)], acc_v)
    gather.wait()

    def acc_body(g, carry):
        s0 = pl.multiple_of(g * 16, 16)
        acc = acc_v[pl.ds(s0, 16)]
        f0 = pl.multiple_of(g * (_PC * 16), 16)
        for q in range(_PC):
            acc = acc + pval_v[pl.ds(f0 + q * 16, 16)]
        acc_v[pl.ds(s0, 16)] = acc
        return carry

    lax.fori_loop(0, _G, acc_body, 0)

    pltpu.sync_copy(acc_v, out.at[pl.ds(base, _BPW)])


def kernel(x, theta0, theta_singles, theta_pairs):
    xT = jnp.asarray(x, jnp.int32).T
    t0v = jnp.broadcast_to(jnp.asarray(theta0, jnp.float32), (16,))
    singles = jnp.asarray(theta_singles, jnp.float32).reshape(-1)
    acc, pidx = _gam_phase1(xT, t0v, singles)
    for c in range(_C):
        chunk_flat = theta_pairs[c * _PC:(c + 1) * _PC].reshape(-1)
        pidx_c = pidx[c * (_NW * _CPW):(c + 1) * (_NW * _CPW)]
        acc = _gam_chunk(chunk_flat, pidx_c, acc)
    return acc


# speedup vs baseline: 1.0476x; 1.0476x over previous
import functools

import jax
import jax.numpy as jnp
from jax import lax
from jax.experimental import pallas as pl
from jax.experimental.pallas import tpu as pltpu
from jax.experimental.pallas import tpu_sc as plsc

_I = 1000
_D = 26
_B = 16384
_PA = (0, 2, 4, 6, 8, 10, 12, 14, 16, 18, 20, 22, 24, 0, 1, 4)
_PB = (1, 3, 5, 7, 9, 11, 13, 15, 17, 19, 21, 23, 25, 2, 3, 6)
_P = 16
_C = 4
_PC = _P // _C

_NC = 2
_NS = 16
_NW = _NC * _NS
_BPW = _B // _NW
_G = _BPW // 16
_PB_W = _P * _BPW
_CPW = _PC * _BPW

_mesh = plsc.VectorSubcoreMesh(
    core_axis_name="c", subcore_axis_name="s", num_cores=_NC, num_subcores=_NS
)


@functools.partial(
    pl.kernel,
    mesh=_mesh,
    out_type=(
        jax.ShapeDtypeStruct((_B,), jnp.float32),
        jax.ShapeDtypeStruct((_B * _P,), jnp.int32),
    ),
    compiler_params=pltpu.CompilerParams(needs_layout_passes=False),
    scratch_types=[
        pltpu.VMEM((_D, _BPW), jnp.int32),
        pltpu.VMEM((_D * _I,), jnp.float32),
        pltpu.VMEM((_PB_W,), jnp.int32),
        pltpu.VMEM((_BPW,), jnp.float32),
        pltpu.VMEM((16,), jnp.float32),
    ],
)
def _gam_phase1(xT, t0, singles, acc_out, pidx_out, x_v, sing_v, pidx_v,
                acc_v, t0_v):
    wid = lax.axis_index("s") * _NC + lax.axis_index("c")
    base = wid * _BPW
    pltpu.sync_copy(xT.at[:, pl.ds(base, _BPW)], x_v)
    pltpu.sync_copy(singles, sing_v)
    pltpu.sync_copy(t0, t0_v)

    def idx_body(g, carry):
        s0 = pl.multiple_of(g * 16, 16)
        acc = t0_v[...]
        for d in range(_D):
            iv = x_v[d, pl.ds(s0, 16)]
            acc = acc + plsc.load_gather(sing_v, [iv + d * _I])
        acc_v[pl.ds(s0, 16)] = acc
        f0 = pl.multiple_of(g * (_PC * 16), 16)
        for p in range(_P):
            i = x_v[_PA[p], pl.ds(s0, 16)]
            j = x_v[_PB[p], pl.ds(s0, 16)]
            pidx_v[pl.ds((p // _PC) * _CPW + f0 + (p % _PC) * 16, 16)] = (
                i * _I + j + (p % _PC) * (_I * _I)
            )
        return carry

    lax.fori_loop(0, _G, idx_body, 0)

    pltpu.sync_copy(acc_v, acc_out.at[pl.ds(base, _BPW)])
    for c in range(_C):
        pltpu.sync_copy(
            pidx_v.at[pl.ds(c * _CPW, _CPW)],
            pidx_out.at[pl.ds((c * _NW + wid) * _CPW, _CPW)],
        )


@functools.partial(
    pl.kernel,
    mesh=_mesh,
    out_type=jax.ShapeDtypeStruct((_B,), jnp.float32),
    compiler_params=pltpu.CompilerParams(needs_layout_passes=False),
    scratch_types=[
        pltpu.VMEM((_CPW,), jnp.int32),
        pltpu.VMEM((_CPW,), jnp.float32),
        pltpu.VMEM((_BPW,), jnp.float32),
        pltpu.SemaphoreType.DMA,
    ],
)
def _gam_chunk(chunk_flat, pidx_c, acc_in, out, pidx_v, pval_v, acc_v, sem):
    wid = lax.axis_index("s") * _NC + lax.axis_index("c")
    base = wid * _BPW
    pltpu.sync_copy(pidx_c.at[pl.ds(wid * _CPW, _CPW)], pidx_v)
    gather = pltpu.async_copy(chunk_flat.at[pidx_v], pval_v, sem)
    pltpu.sync_copy(acc_in.at[pl.ds(base, _BPW)], acc_v)
    gather.wait()

    def acc_body(g, carry):
        s0 = pl.multiple_of(g * 16, 16)
        acc = acc_v[pl.ds(s0, 16)]
        f0 = pl.multiple_of(g * (_PC * 16), 16)
        for q in range(_PC):
            acc = acc + pval_v[pl.ds(f0 + q * 16, 16)]
        acc_v[pl.ds(s0, 16)] = acc
        return carry

    lax.fori_loop(0, _G, acc_body, 0)

    pltpu.sync_copy(acc_v, out.at[pl.ds(base, _BPW)])


def kernel(x, theta0, theta_singles, theta_pairs):
    xT = jnp.asarray(x, jnp.int32).T
    t0v = jnp.broadcast_to(jnp.asarray(theta0, jnp.float32), (16,))
    singles = jnp.asarray(theta_singles, jnp.float32).reshape(-1)
    acc, pidx = _gam_phase1(xT, t0v, singles)
    for c in range(_C):
        chunk_flat = jax.lax.optimization_barrier(
            theta_pairs[c * _PC:(c + 1) * _PC].reshape(-1))
        pidx_c = pidx[c * (_NW * _CPW):(c + 1) * (_NW * _CPW)]
        acc = _gam_chunk(chunk_flat, pidx_c, acc)
    return acc


# speedup vs baseline: 1.4677x; 1.4011x over previous
import functools

import jax
import jax.numpy as jnp
from jax import lax
from jax.experimental import pallas as pl
from jax.experimental.pallas import tpu as pltpu
from jax.experimental.pallas import tpu_sc as plsc

_I = 1000
_D = 26
_B = 16384
_PA = (0, 2, 4, 6, 8, 10, 12, 14, 16, 18, 20, 22, 24, 0, 1, 4)
_PB = (1, 3, 5, 7, 9, 11, 13, 15, 17, 19, 21, 23, 25, 2, 3, 6)
_P = 16

_NC = 2
_NS = 16
_NW = _NC * _NS
_BPW = _B // _NW
_G = _BPW // 16
_PB_W = _P * _BPW

_mesh = plsc.VectorSubcoreMesh(
    core_axis_name="c", subcore_axis_name="s", num_cores=_NC, num_subcores=_NS
)


@functools.partial(
    pl.kernel,
    mesh=_mesh,
    out_type=(
        jax.ShapeDtypeStruct((_B,), jnp.float32),
        jax.ShapeDtypeStruct((_B * _P,), jnp.int32),
    ),
    compiler_params=pltpu.CompilerParams(needs_layout_passes=False),
    scratch_types=[
        pltpu.VMEM((_D, _BPW), jnp.int32),
        pltpu.VMEM((_D * _I,), jnp.float32),
        pltpu.VMEM((_PB_W,), jnp.int32),
        pltpu.VMEM((_BPW,), jnp.float32),
        pltpu.VMEM((16,), jnp.float32),
    ],
)
def _gam_phase1(xT, t0, singles, acc_out, pidx_out, x_v, sing_v, pidx_v,
                acc_v, t0_v):
    wid = lax.axis_index("s") * _NC + lax.axis_index("c")
    base = wid * _BPW
    pltpu.sync_copy(xT.at[:, pl.ds(base, _BPW)], x_v)
    pltpu.sync_copy(singles, sing_v)
    pltpu.sync_copy(t0, t0_v)

    def idx_body(g, carry):
        s0 = pl.multiple_of(g * 16, 16)
        acc = t0_v[...]
        for d in range(_D):
            iv = x_v[d, pl.ds(s0, 16)]
            acc = acc + plsc.load_gather(sing_v, [iv + d * _I])
        acc_v[pl.ds(s0, 16)] = acc
        f0 = pl.multiple_of(g * (_P * 16), 16)
        for p in range(_P):
            i = x_v[_PA[p], pl.ds(s0, 16)]
            j = x_v[_PB[p], pl.ds(s0, 16)]
            pidx_v[pl.ds(f0 + p * 16, 16)] = i * _I + j + p * (_I * _I)
        return carry

    lax.fori_loop(0, _G, idx_body, 0)

    pltpu.sync_copy(acc_v, acc_out.at[pl.ds(base, _BPW)])
    pltpu.sync_copy(pidx_v, pidx_out.at[pl.ds(wid * _PB_W, _PB_W)])


@functools.partial(
    pl.kernel,
    mesh=_mesh,
    out_type=jax.ShapeDtypeStruct((_B,), jnp.float32),
    compiler_params=pltpu.CompilerParams(needs_layout_passes=False),
    scratch_types=[
        pltpu.VMEM((_PB_W,), jnp.int32),
        pltpu.VMEM((_PB_W,), jnp.float32),
        pltpu.VMEM((_BPW,), jnp.float32),
        pltpu.SemaphoreType.DMA,
    ],
)
def _gam_phase2(pairs, pidx_hbm, acc_hbm, out, pidx_v, pval_v, acc_v, sem):
    wid = lax.axis_index("s") * _NC + lax.axis_index("c")
    base = wid * _BPW
    pltpu.sync_copy(pidx_hbm.at[pl.ds(wid * _PB_W, _PB_W)], pidx_v)
    gather = pltpu.async_copy(pairs.at[pidx_v], pval_v, sem)
    pltpu.sync_copy(acc_hbm.at[pl.ds(base, _BPW)], acc_v)
    gather.wait()

    def acc_body(g, carry):
        s0 = pl.multiple_of(g * 16, 16)
        acc = acc_v[pl.ds(s0, 16)]
        f0 = pl.multiple_of(g * (_P * 16), 16)
        for p in range(_P):
            acc = acc + pval_v[pl.ds(f0 + p * 16, 16)]
        acc_v[pl.ds(s0, 16)] = acc
        return carry

    lax.fori_loop(0, _G, acc_body, 0)

    pltpu.sync_copy(acc_v, out.at[pl.ds(base, _BPW)])


def kernel(x, theta0, theta_singles, theta_pairs):
    xT = jnp.asarray(x, jnp.int32).T
    t0v = jnp.broadcast_to(jnp.asarray(theta0, jnp.float32), (16,))
    singles = jnp.asarray(theta_singles, jnp.float32).reshape(-1)
    pairs = jnp.asarray(theta_pairs, jnp.float32).reshape(-1)
    acc, pidx = _gam_phase1(xT, t0v, singles)
    return _gam_phase2(pairs, pidx, acc)


# speedup vs baseline: 1.4714x; 1.0025x over previous
import functools

import jax
import jax.numpy as jnp
from jax import lax
from jax.experimental import pallas as pl
from jax.experimental.pallas import tpu as pltpu
from jax.experimental.pallas import tpu_sc as plsc

_I = 1000
_D = 26
_B = 16384
_PA = (0, 2, 4, 6, 8, 10, 12, 14, 16, 18, 20, 22, 24, 0, 1, 4)
_PB = (1, 3, 5, 7, 9, 11, 13, 15, 17, 19, 21, 23, 25, 2, 3, 6)
_P = 16

_NC = 2
_NS = 16
_NW = _NC * _NS
_BPW = _B // _NW
_G = _BPW // 16
_PB_W = _P * _BPW

_mesh = plsc.VectorSubcoreMesh(
    core_axis_name="c", subcore_axis_name="s", num_cores=_NC, num_subcores=_NS
)


@functools.partial(
    pl.kernel,
    mesh=_mesh,
    out_type=(
        jax.ShapeDtypeStruct((_B,), jnp.float32),
        jax.ShapeDtypeStruct((_B * _P,), jnp.int32),
    ),
    compiler_params=pltpu.CompilerParams(needs_layout_passes=False),
    scratch_types=[
        pltpu.VMEM((_D, _BPW), jnp.int32),
        pltpu.VMEM((_D * _I,), jnp.float32),
        pltpu.VMEM((_PB_W,), jnp.int32),
        pltpu.VMEM((_BPW,), jnp.float32),
        pltpu.VMEM((16,), jnp.float32),
    ],
)
def _gam_phase1(xT, t0, singles, acc_out, pidx_out, x_v, sing_v, pidx_v,
                acc_v, t0_v):
    wid = lax.axis_index("s") * _NC + lax.axis_index("c")
    base = wid * _BPW
    pltpu.sync_copy(xT.at[:, pl.ds(base, _BPW)], x_v)
    pltpu.sync_copy(singles, sing_v)
    pltpu.sync_copy(t0, t0_v)

    def idx_body(g, carry):
        s0 = pl.multiple_of(g * 16, 16)
        acc = t0_v[...]
        for d in range(_D):
            iv = x_v[d, pl.ds(s0, 16)]
            acc = acc + plsc.load_gather(sing_v, [iv + d * _I])
        acc_v[pl.ds(s0, 16)] = acc
        f0 = pl.multiple_of(g * (_P * 16), 16)
        for p in range(_P):
            i = x_v[_PA[p], pl.ds(s0, 16)]
            j = x_v[_PB[p], pl.ds(s0, 16)]
            pidx_v[pl.ds(f0 + p * 16, 16)] = i * _I + j + p * (_I * _I)
        return carry

    lax.fori_loop(0, _G, idx_body, 0)

    pltpu.sync_copy(acc_v, acc_out.at[pl.ds(base, _BPW)])
    pltpu.sync_copy(pidx_v, pidx_out.at[pl.ds(wid * _PB_W, _PB_W)])


@functools.partial(
    pl.kernel,
    mesh=_mesh,
    out_type=jax.ShapeDtypeStruct((_B,), jnp.float32),
    compiler_params=pltpu.CompilerParams(needs_layout_passes=False),
    scratch_types=[
        pltpu.VMEM((_PB_W,), jnp.int32),
        pltpu.VMEM((_PB_W,), jnp.float32),
        pltpu.VMEM((_BPW,), jnp.float32),
        pltpu.SemaphoreType.DMA,
        pltpu.SemaphoreType.DMA,
    ],
)
def _gam_phase2(pairs, pidx_hbm, acc_hbm, out, pidx_v, pval_v, acc_v, sem0,
                sem1):
    wid = lax.axis_index("s") * _NC + lax.axis_index("c")
    base = wid * _BPW
    _H = _PB_W // 2
    pltpu.sync_copy(pidx_hbm.at[pl.ds(wid * _PB_W, _PB_W)], pidx_v)
    g0 = pltpu.async_copy(pairs.at[pidx_v.at[pl.ds(0, _H)]],
                          pval_v.at[pl.ds(0, _H)], sem0)
    g1 = pltpu.async_copy(pairs.at[pidx_v.at[pl.ds(_H, _H)]],
                          pval_v.at[pl.ds(_H, _H)], sem1)
    pltpu.sync_copy(acc_hbm.at[pl.ds(base, _BPW)], acc_v)

    def acc_body(g, carry):
        s0 = pl.multiple_of(g * 16, 16)
        acc = acc_v[pl.ds(s0, 16)]
        f0 = pl.multiple_of(g * (_P * 16), 16)
        for p in range(_P):
            acc = acc + pval_v[pl.ds(f0 + p * 16, 16)]
        acc_v[pl.ds(s0, 16)] = acc
        return carry

    g0.wait()
    lax.fori_loop(0, _G // 2, acc_body, 0)
    g1.wait()
    lax.fori_loop(_G // 2, _G, acc_body, 0)

    pltpu.sync_copy(acc_v, out.at[pl.ds(base, _BPW)])


def kernel(x, theta0, theta_singles, theta_pairs):
    xT = jnp.asarray(x, jnp.int32).T
    t0v = jnp.broadcast_to(jnp.asarray(theta0, jnp.float32), (16,))
    singles = jnp.asarray(theta_singles, jnp.float32).reshape(-1)
    pairs = jnp.asarray(theta_pairs, jnp.float32).reshape(-1)
    acc, pidx = _gam_phase1(xT, t0v, singles)
    return _gam_phase2(pairs, pidx, acc)
